# Initial kernel scaffold; baseline (speedup 1.0000x reference)
import jax
import jax.numpy as jnp
from jax.experimental import pallas as pl


def kernel(idx_0, idx_1, idx_2, idx_3, idx_4, idx_5, idx_6, idx_7, idx_8, W_0, W_1, W_2, W_3, W_4, W_5, W_6, W_7, W_8):
    raise NotImplementedError("write your pallas kernel here")


# speedup vs baseline: 9.5258x; 9.5258x over previous
import functools

import jax
import jax.numpy as jnp
from jax import lax
from jax.experimental import pallas as pl
from jax.experimental.pallas import tpu as pltpu
from jax.experimental.pallas import tpu_sc as plsc

_BINS = (64, 256, 64, 256, 512, 256, 512, 512, 256)
_OFF = tuple(sum(_BINS[:i]) for i in range(len(_BINS)))
_D = 8
_B = 16384
_F = len(_BINS)

_NC = 2
_NS = 16
_NW = _NC * _NS
_C = _B // _NW
_R = _C * _F
_GCHUNK = 128
_NG = _R // _GCHUNK


def _body(i0, i1, i2, i3, i4, i5, i6, i7, i8, tab, out, idx_v, gidx, rows, sem):
    idx_refs = (i0, i1, i2, i3, i4, i5, i6, i7, i8)
    cid = lax.axis_index("c")
    sid = lax.axis_index("s")
    wid = sid * _NC + cid
    base = wid * _C

    for f in range(_F):
        pltpu.sync_copy(idx_refs[f].at[pl.ds(base, _C)], idx_v.at[pl.ds(f * _C, _C)])

    nine_iota = lax.iota(jnp.int32, 16) * 9

    def interleave(j, carry):
        for f in range(_F):
            v = idx_v[pl.ds(f * _C + j * 16, 16)]
            v = (v & (_BINS[f] - 1)) + _OFF[f]
            pos = nine_iota + (j * (16 * _F) + f)
            plsc.store_scatter(gidx, [pos], v)
        return carry

    lax.fori_loop(0, _C // 16, interleave, 0, unroll=False)

    def fire(j, carry):
        pltpu.async_copy(
            tab.at[gidx.at[pl.ds(j * _GCHUNK, _GCHUNK)]],
            rows.at[pl.ds(j * _GCHUNK, _GCHUNK)],
            sem,
        )
        return carry

    lax.fori_loop(0, _NG, fire, 0, unroll=False)

    def drain(j, carry):
        pltpu.make_async_copy(
            tab.at[gidx.at[pl.ds(j * _GCHUNK, _GCHUNK)]],
            rows.at[pl.ds(j * _GCHUNK, _GCHUNK)],
            sem,
        ).wait()
        return carry

    lax.fori_loop(0, _NG, drain, 0, unroll=False)

    pltpu.sync_copy(rows, out.at[pl.ds(wid * _R, _R)])


@jax.jit
def kernel(idx_0, idx_1, idx_2, idx_3, idx_4, idx_5, idx_6, idx_7, idx_8,
           W_0, W_1, W_2, W_3, W_4, W_5, W_6, W_7, W_8):
    tab = jnp.concatenate([W_0, W_1, W_2, W_3, W_4, W_5, W_6, W_7, W_8], axis=0)
    mesh = plsc.VectorSubcoreMesh(
        core_axis_name="c", subcore_axis_name="s", num_cores=_NC, num_subcores=_NS
    )
    run = pl.kernel(
        _body,
        out_type=jax.ShapeDtypeStruct((_B * _F, _D), jnp.float32),
        mesh=mesh,
        scratch_types=[
            pltpu.VMEM((_F * _C,), jnp.int32),
            pltpu.VMEM((_R,), jnp.int32),
            pltpu.VMEM((_R, _D), jnp.float32),
            pltpu.SemaphoreType.DMA,
        ],
        compiler_params=pltpu.CompilerParams(
            needs_layout_passes=False, use_tc_tiling_on_sc=False
        ),
    )
    flat = run(idx_0, idx_1, idx_2, idx_3, idx_4, idx_5, idx_6, idx_7, idx_8, tab)
    return flat.reshape(_B, _F * _D)
